# Initial kernel scaffold; baseline (speedup 1.0000x reference)
import jax
import jax.numpy as jnp
from jax.experimental import pallas as pl


def kernel(feat_src, adj_src, ppmi_src, feat_tgt, adj_tgt, ppmi_tgt, label_src, domain_label, adj_label_src, adj_label_tgt, norm_src, norm_tgt, pos_weight_src, pos_weight_tgt, train_idx, epoch, params):
    raise NotImplementedError("write your pallas kernel here")


# speedup vs baseline: 1.5366x; 1.5366x over previous
import functools

import jax
import jax.numpy as jnp
from jax.experimental import pallas as pl
from jax.experimental.pallas import tpu as pltpu

N = 4096
D_IN = 512
HID = 32
OUT = 16
NC = 8
LMD_D = 0.1
LMD_R = 1.0
LMD_F = 1.0

_BLK = 512


def _p1_kernel(a_ref, f_ref, w1_ref, b1_ref, o_ref, t_ref):
    @pl.when(pl.program_id(0) == 0)
    def _():
        t_ref[...] = jnp.dot(f_ref[...], w1_ref[...],
                             preferred_element_type=jnp.float32)
    y = jnp.dot(a_ref[...], t_ref[...],
                preferred_element_type=jnp.float32) + b1_ref[...]
    o_ref[...] = jnp.concatenate(
        [y[:, :HID], jnp.maximum(y[:, HID:], 0.0)], axis=1)


def _p2_kernel(a_ref, s_ref, wz_ref, b2_ref, o_ref, t_ref):
    @pl.when(pl.program_id(0) == 0)
    def _():
        t_ref[...] = jnp.dot(s_ref[...], wz_ref[...],
                             preferred_element_type=jnp.float32)
    o_ref[...] = jnp.dot(a_ref[...], t_ref[...],
                         preferred_element_type=jnp.float32) + b2_ref[...]


def _adj_pass(a, x, w, b, kernel_fn):
    nb = N // _BLK
    return pl.pallas_call(
        kernel_fn,
        grid=(nb,),
        in_specs=[
            pl.BlockSpec((_BLK, N), lambda i: (i, 0)),
            pl.BlockSpec(x.shape, lambda i: (0, 0)),
            pl.BlockSpec(w.shape, lambda i: (0, 0)),
            pl.BlockSpec(b.shape, lambda i: (0, 0)),
        ],
        out_specs=pl.BlockSpec((_BLK, 2 * HID), lambda i: (i, 0)),
        out_shape=jax.ShapeDtypeStruct((N, 2 * HID), jnp.float32),
        scratch_shapes=[pltpu.VMEM((N, 2 * HID), jnp.float32)],
    )(a, x, w, b)


def _bce_kernel(zb_ref, z_ref, y_ref, pw_ref, o_ref):
    x = jax.lax.dot_general(zb_ref[...], z_ref[...],
                            (((1,), (1,)), ((), ())),
                            preferred_element_type=jnp.float32)
    sp = jnp.maximum(-x, 0.0) + jnp.log1p(jnp.exp(-jnp.abs(x)))
    y = y_ref[...]
    pw = pw_ref[0, 0]
    part = jnp.sum(pw * y * sp + (1.0 - y) * (x + sp))

    @pl.when(pl.program_id(0) == 0)
    def _():
        o_ref[...] = jnp.zeros_like(o_ref)
    o_ref[...] = o_ref[...] + jnp.reshape(part, (1, 1))


def _bce_sum(z, label, pw):
    nb = N // _BLK
    return pl.pallas_call(
        _bce_kernel,
        grid=(nb,),
        in_specs=[
            pl.BlockSpec((_BLK, z.shape[1]), lambda i: (i, 0)),
            pl.BlockSpec(z.shape, lambda i: (0, 0)),
            pl.BlockSpec((_BLK, N), lambda i: (i, 0)),
            pl.BlockSpec((1, 1), lambda i: (0, 0)),
        ],
        out_specs=pl.BlockSpec((1, 1), lambda i: (0, 0)),
        out_shape=jax.ShapeDtypeStruct((1, 1), jnp.float32),
    )(z, z, label, pw.reshape(1, 1))[0, 0]


def _att(f1, f2, W, b):
    st = jnp.stack([f1, f2], axis=1)
    w = jax.nn.softmax(st @ W + b, axis=1)
    return jnp.sum(st * w, axis=1)


def _diff(a, b):
    na = jnp.linalg.norm(a, axis=1, keepdims=True)
    nb = jnp.linalg.norm(b, axis=1, keepdims=True)
    a2 = a / (na + 1e-6)
    b2 = b / (nb + 1e-6)
    return jnp.mean((a2.T @ b2) ** 2)


def _xent(logits, labels):
    lse = jax.nn.logsumexp(logits, axis=1)
    ll = jnp.take_along_axis(logits, labels[:, None], axis=1)[:, 0]
    return jnp.mean(lse - ll)


def _kld(mu, lv, num_nodes):
    return -0.5 / num_nodes * jnp.mean(
        jnp.sum(1.0 + 2.0 * lv - mu ** 2 - jnp.exp(lv) ** 2, axis=1))


def _encode_domain(feat, adj, ppmi, p, pre_p_l, pre_p_g, pre_s_l, pre_s_g):
    out = {}
    for a, pre_vae, pre_gcn in ((adj, pre_p_l, pre_s_l),
                                (ppmi, pre_p_g, pre_s_g)):
        w1 = jnp.concatenate([p[pre_vae + '_gc1_W'], p[pre_gcn + '_gc1_W']],
                             axis=1)
        b1 = jnp.concatenate([p[pre_vae + '_gc1_b'], p[pre_gcn + '_gc1_b']]
                             )[None, :]
        s = _adj_pass(a, feat, w1, b1, _p1_kernel)

        wz = jnp.zeros((2 * HID, 4 * OUT), jnp.float32)
        wz = wz.at[:HID, :2 * OUT].set(
            jnp.concatenate([p[pre_vae + '_gc2_W'], p[pre_vae + '_gc3_W']],
                            axis=1))
        wz = wz.at[HID:, 2 * OUT:].set(
            jnp.concatenate([p[pre_gcn + '_gc2_W'], p[pre_gcn + '_gc3_W']],
                            axis=1))
        b2 = jnp.concatenate([p[pre_vae + '_gc2_b'], p[pre_vae + '_gc3_b'],
                              p[pre_gcn + '_gc2_b'], p[pre_gcn + '_gc3_b']]
                             )[None, :]
        r = _adj_pass(a, s, wz, b2, _p2_kernel)
        out[pre_vae] = (r[:, :OUT], r[:, OUT:2 * OUT])
        out[pre_gcn] = (r[:, 2 * OUT:3 * OUT], r[:, 3 * OUT:])
    return out


def kernel(feat_src, adj_src, ppmi_src, feat_tgt, adj_tgt, ppmi_tgt,
           label_src, domain_label, adj_label_src, adj_label_tgt,
           norm_src, norm_tgt, pos_weight_src, pos_weight_tgt,
           train_idx, epoch, params):
    p = params
    enc_s = _encode_domain(feat_src, adj_src, ppmi_src, p,
                           'p_l', 'p_g', 's_l', 's_g')
    enc_t = _encode_domain(feat_tgt, adj_tgt, ppmi_tgt, p,
                           'p_l', 'p_g', 's_l', 's_g')

    emb_s = _att(enc_s['s_l'][0], enc_s['s_g'][0], p['att_W'], p['att_b'])
    emb_t = _att(enc_t['s_l'][0], enc_t['s_g'][0], p['att_W'], p['att_b'])
    emb = jnp.concatenate([emb_s, emb_t], axis=0)

    pred_logit = emb @ p['clf_W'] + p['clf_b']
    h = jax.nn.relu(emb @ p['dd1_W'] + p['dd1_b'])
    d_logit = h @ p['dd2_W'] + p['dd2_b']

    diff_loss = (_diff(enc_s['p_l'][0], enc_s['s_l'][0])
                 + _diff(enc_t['p_l'][0], enc_t['s_l'][0]))
    clf_loss = _xent(pred_logit[train_idx, :], label_src[train_idx])
    dom_loss = _xent(d_logit, domain_label)

    z_s = jnp.concatenate(
        [_att(enc_s['p_l'][0], enc_s['p_g'][0], p['sa_src_W'], p['sa_src_b']),
         _att(enc_s['s_l'][0], enc_s['s_g'][0], p['sa_src_W'], p['sa_src_b'])],
        axis=1)
    z_t = jnp.concatenate(
        [_att(enc_t['p_l'][0], enc_t['p_g'][0], p['sa_tgt_W'], p['sa_tgt_b']),
         _att(enc_t['s_l'][0], enc_t['s_g'][0], p['sa_tgt_W'], p['sa_tgt_b'])],
        axis=1)

    bce_s = _bce_sum(z_s, adj_label_src, pos_weight_src) / (N * N)
    bce_t = _bce_sum(z_t, adj_label_tgt, pos_weight_tgt) / (N * N)

    mu_s = jnp.concatenate([enc_s['p_l'][0], enc_s['p_g'][0],
                            enc_s['s_l'][0], enc_s['s_g'][0]], axis=1)
    lv_s = jnp.concatenate([enc_s['p_l'][1], enc_s['p_g'][1],
                            enc_s['s_l'][1], enc_s['s_g'][1]], axis=1)
    mu_t = jnp.concatenate([enc_t['p_l'][0], enc_t['p_g'][0],
                            enc_t['s_l'][0], enc_t['s_g'][0]], axis=1)
    lv_t = jnp.concatenate([enc_t['p_l'][1], enc_t['p_g'][1],
                            enc_t['s_l'][1], enc_t['s_g'][1]], axis=1)

    recon = (norm_src[0] * bce_s + _kld(mu_s, lv_s, N)
             + norm_tgt[0] * bce_t + _kld(mu_t, lv_t, N))

    total = clf_loss + LMD_D * diff_loss + LMD_F * dom_loss + LMD_R * recon
    return jnp.reshape(total, (1,))
